# baseline (speedup 1.0000x reference)
import functools

import jax
import jax.numpy as jnp
from jax import lax
from jax.experimental import pallas as pl
from jax.experimental.pallas import tpu as pltpu
from jax.experimental.pallas import tpu_sc as plsc

NC, NS = 2, 16
CH, CW = 16, 16

HID = 256
EMBD = 512
SYMV = 10000
SYMD = 64
NFEAT = {"clause": 7, "literal": 3, "term": 8, "symbol": 6, "variable": 1}
NN = {"clause": 10000, "literal": 30000, "term": 60000, "symbol": 10000, "variable": 15000}
ETYPES = [
    ("contains_literal", False, "clause", "literal", "contains_literal"),
    ("has_atom", False, "literal", "term", "has_atom"),
    ("has_arg", False, "term", "term", "has_arg"),
    ("symbol_of", False, "term", "symbol", "symbol_of"),
    ("var_occurrence", False, "variable", "term", "var_occurrence"),
    ("shared_variable", False, "variable", "variable", "shared_variable"),
    ("contains_literal", True, "literal", "clause", "rev_contains_literal"),
    ("has_atom", True, "term", "literal", "rev_has_atom"),
    ("has_arg", True, "term", "term", "rev_has_arg"),
    ("symbol_of", True, "symbol", "term", "rev_symbol_of"),
    ("var_occurrence", True, "term", "variable", "rev_var_occurrence"),
]
RBLK = 1000


def _proj_body(x_ref, w_ref, b_ref, o_ref):
    o_ref[...] = jax.nn.relu(
        jnp.dot(x_ref[...], w_ref[...], preferred_element_type=jnp.float32)
        + b_ref[...]
    )


def _in_proj(x, w, b):
    n, f = x.shape
    fp = 8
    xp = jnp.pad(x, ((0, 0), (0, fp - f)))
    wp = jnp.pad(w, ((0, fp - f), (0, 0)))
    return pl.pallas_call(
        _proj_body,
        grid=(n // RBLK,),
        in_specs=[
            pl.BlockSpec((RBLK, fp), lambda i: (i, 0)),
            pl.BlockSpec((fp, HID), lambda i: (0, 0)),
            pl.BlockSpec((1, HID), lambda i: (0, 0)),
        ],
        out_specs=pl.BlockSpec((RBLK, HID), lambda i: (i, 0)),
        out_shape=jax.ShapeDtypeStruct((n, HID), jnp.float32),
    )(xp, wp, b.reshape(1, HID))


def _symcomb_body(x_ref, e_ref, w1_ref, w2_ref, b_ref, o_ref):
    o_ref[...] = jax.nn.relu(
        jnp.dot(x_ref[...], w1_ref[...], preferred_element_type=jnp.float32)
        + jnp.dot(e_ref[...], w2_ref[...], preferred_element_type=jnp.float32)
        + b_ref[...]
    )


def _symcomb(x, emb, wc, bc):
    n = x.shape[0]
    return pl.pallas_call(
        _symcomb_body,
        grid=(n // RBLK,),
        in_specs=[
            pl.BlockSpec((RBLK, HID), lambda i: (i, 0)),
            pl.BlockSpec((RBLK, SYMD), lambda i: (i, 0)),
            pl.BlockSpec((HID, HID), lambda i: (0, 0)),
            pl.BlockSpec((SYMD, HID), lambda i: (0, 0)),
            pl.BlockSpec((1, HID), lambda i: (0, 0)),
        ],
        out_specs=pl.BlockSpec((RBLK, HID), lambda i: (i, 0)),
        out_shape=jax.ShapeDtypeStruct((n, HID), jnp.float32),
    )(x, emb, wc[:HID], wc[HID:], bc.reshape(1, HID))


def _update_body(k, x_ref, *refs):
    s_refs = refs[:k]
    c_refs = refs[k : 2 * k]
    wl_ref, wr_ref, bl_ref, g_ref, b_ref, o_ref = refs[2 * k :]
    xb = x_ref[...]
    wr_sum = jnp.sum(wr_ref[...], axis=0)
    o = jnp.dot(xb, wr_sum, preferred_element_type=jnp.float32)
    o = o + jnp.sum(bl_ref[...], axis=0)[None, :]
    for i in range(k):
        ic = 1.0 / jnp.maximum(c_refs[i][...], 1.0)
        o = o + jnp.dot(
            s_refs[i][...] * ic, wl_ref[i], preferred_element_type=jnp.float32
        )
    h = o + xb
    m = jnp.mean(h, axis=-1, keepdims=True)
    v = jnp.mean((h - m) ** 2, axis=-1, keepdims=True)
    o_ref[...] = (h - m) * jax.lax.rsqrt(v + 1e-5) * g_ref[...] + b_ref[...]


def _update(x, s_list, cnt_list, wl, wr, bl, g, b):
    n = x.shape[0]
    k = len(s_list)
    in_specs = [pl.BlockSpec((RBLK, HID), lambda i: (i, 0))]
    in_specs += [pl.BlockSpec((RBLK, HID), lambda i: (i, 0))] * k
    in_specs += [pl.BlockSpec((RBLK, 1), lambda i: (i, 0))] * k
    in_specs += [
        pl.BlockSpec((k, HID, HID), lambda i: (0, 0, 0)),
        pl.BlockSpec((k, HID, HID), lambda i: (0, 0, 0)),
        pl.BlockSpec((k, HID), lambda i: (0, 0)),
        pl.BlockSpec((1, HID), lambda i: (0, 0)),
        pl.BlockSpec((1, HID), lambda i: (0, 0)),
    ]
    return pl.pallas_call(
        functools.partial(_update_body, k),
        grid=(n // RBLK,),
        in_specs=in_specs,
        out_specs=pl.BlockSpec((RBLK, HID), lambda i: (i, 0)),
        out_shape=jax.ShapeDtypeStruct((n, HID), jnp.float32),
    )(x, *s_list, *cnt_list, wl, wr, bl, g.reshape(1, HID), b.reshape(1, HID))


def _head_body(x_ref, w1_ref, b1_ref, w2_ref, b2_ref, o_ref):
    h = jax.nn.relu(
        jnp.dot(x_ref[...], w1_ref[...], preferred_element_type=jnp.float32)
        + b1_ref[...]
    )
    o_ref[...] = (
        jnp.dot(h, w2_ref[...], preferred_element_type=jnp.float32) + b2_ref[...]
    )


def _head(x, w1, b1, w2, b2):
    n = x.shape[0]
    return pl.pallas_call(
        _head_body,
        grid=(n // RBLK,),
        in_specs=[
            pl.BlockSpec((RBLK, HID), lambda i: (i, 0)),
            pl.BlockSpec((HID, HID), lambda i: (0, 0)),
            pl.BlockSpec((1, HID), lambda i: (0, 0)),
            pl.BlockSpec((HID, EMBD), lambda i: (0, 0)),
            pl.BlockSpec((1, EMBD), lambda i: (0, 0)),
        ],
        out_specs=pl.BlockSpec((RBLK, EMBD), lambda i: (i, 0)),
        out_shape=jax.ShapeDtypeStruct((n, EMBD), jnp.float32),
    )(x, w1, b1.reshape(1, HID), w2, b2.reshape(1, EMBD))


def _rup(x, m):
    return (x + m - 1) // m * m


NTS = list(NN)


def _zero_fill(buf, val):
    def zf(i, _):
        for k in range(CW // 16):
            buf[i, pl.ds(k * 16, 16)] = jnp.full((16,), val, jnp.float32)
        return 0

    lax.fori_loop(0, 128, zf, 0)


def _my_blocks(sid, nzt):
    return (nzt - sid + NS - 1) // NS


def _fill_rows(src_v, dst_ref, sid, nzt):
    def zb(j, _):
        blk = sid + j * NS
        pltpu.sync_copy(src_v, dst_ref.at[pl.ds(blk * 128, 128)])
        return 0

    lax.fori_loop(0, _my_blocks(sid, nzt), zb, 0)


def _sc_prepass_body(meta, n_emb_rows, *refs):
    num_et = len(meta)
    table, symidx = refs[0], refs[1]
    dst2 = refs[2 : 2 + num_et]
    emb_out = refs[2 + num_et]
    cnt_out = refs[3 + num_et : 3 + 2 * num_et]
    acc, dstv, zerosv, onesv, rows64, sg, ss0, ss1 = refs[3 + 2 * num_et :]
    cid = lax.axis_index("c")
    sid = lax.axis_index("s")
    _zero_fill(zerosv, 0.0)
    _zero_fill(onesv, 1.0)

    @pl.when(cid == 0)
    def _():
        rpt = n_emb_rows // NS
        pltpu.sync_copy(symidx.at[sid], dstv.at[pl.ds(0, rpt)])

        def eb(j, _):
            r = sid * rpt + j
            pltpu.async_copy(table.at[dstv.at[j]], rows64, sg).wait()
            pltpu.sync_copy(rows64, emb_out.at[pl.ds(r * 128, 128)])
            return 0

        lax.fori_loop(0, rpt, eb, 0)

    for t in range(num_et):
        m = meta[t]
        nb, npp = m["nb"], m["np_pad"]
        nzt = npp // 128

        @pl.when(cid == t % 2)
        def _(t=t, nb=nb, nzt=nzt):
            _fill_rows(zerosv, acc, sid, nzt)
            pltpu.sync_copy(dst2[t].at[sid], dstv.at[pl.ds(0, nb)])
            plsc.subcore_barrier()
            pltpu.async_copy(onesv, acc.at[dstv.at[0]], ss0, add=True)
            pltpu.async_copy(onesv, acc.at[dstv.at[1]], ss1, add=True)

            def rb(jj, _):
                pltpu.make_async_copy(onesv, acc.at[dstv.at[0]], ss0).wait()
                pltpu.async_copy(onesv, acc.at[dstv.at[2 * jj]], ss0, add=True)
                pltpu.make_async_copy(onesv, acc.at[dstv.at[0]], ss1).wait()
                pltpu.async_copy(
                    onesv, acc.at[dstv.at[2 * jj + 1]], ss1, add=True
                )
                return 0

            lax.fori_loop(1, nb // 2, rb, 0)
            pltpu.make_async_copy(onesv, acc.at[dstv.at[0]], ss0).wait()
            pltpu.make_async_copy(onesv, acc.at[dstv.at[0]], ss1).wait()
            plsc.subcore_barrier()

            def co(j, _):
                r0 = (sid + j * NS) * 128
                pltpu.sync_copy(
                    acc.at[pl.ds(r0, 128)], cnt_out[t].at[pl.ds(r0, 128)]
                )
                return 0

            lax.fori_loop(0, _my_blocks(sid, nzt), co, 0)


def _sc_layer_body(meta, *refs):
    num_et = len(meta)
    ei_refs = refs[: 2 * num_et]
    x8 = refs[2 * num_et : 2 * num_et + len(NTS)]
    outs = refs[2 * num_et + len(NTS) : 3 * num_et + len(NTS)]
    (acc, srcv, dstv, idxv, rows0, rows1, zerosv,
     sg0, sg1, ss0, ss1) = refs[3 * num_et + len(NTS) :]
    cid = lax.axis_index("c")
    sid = lax.axis_index("s")
    _zero_fill(zerosv, 0.0)

    ramp8 = lax.iota(jnp.int32, 16) * CH

    for t in range(num_et):
        m = meta[t]
        nb, npp = m["nb"], m["np_pad"]
        src2, dst2 = ei_refs[2 * t], ei_refs[2 * t + 1]
        xsrc = x8[m["src_i"]]
        out = outs[t]
        nzt = npp // 128
        pltpu.sync_copy(src2.at[sid], srcv.at[pl.ds(0, nb)])
        pltpu.sync_copy(dst2.at[sid], dstv.at[pl.ds(0, nb)])

        def chunk_body(cc, _, nb=nb, nzt=nzt, xsrc=xsrc, out=out):
            c = cid * (CH // NC) + cc
            _fill_rows(zerosv, acc, sid, nzt)

            def ib(j, _):
                for k in range(8):
                    v = srcv[j, pl.ds(k * 16, 16)]
                    idxv[j, pl.ds(k * 16, 16)] = v * CH + c
                return 0

            lax.fori_loop(0, nb, ib, 0)
            plsc.subcore_barrier()
            g0 = pltpu.async_copy(xsrc.at[idxv.at[0]], rows0, sg0)
            g1 = pltpu.async_copy(xsrc.at[idxv.at[1]], rows1, sg1)
            g0.wait()
            pltpu.async_copy(rows0, acc.at[dstv.at[0]], ss0, add=True)
            g1.wait()
            pltpu.async_copy(rows1, acc.at[dstv.at[1]], ss1, add=True)

            def rb(jj, _):
                j0, j1 = 2 * jj, 2 * jj + 1
                pltpu.make_async_copy(rows0, acc.at[dstv.at[0]], ss0).wait()
                ga = pltpu.async_copy(xsrc.at[idxv.at[j0]], rows0, sg0)
                pltpu.make_async_copy(rows1, acc.at[dstv.at[0]], ss1).wait()
                gb = pltpu.async_copy(xsrc.at[idxv.at[j1]], rows1, sg1)
                ga.wait()
                pltpu.async_copy(rows0, acc.at[dstv.at[j0]], ss0, add=True)
                gb.wait()
                pltpu.async_copy(rows1, acc.at[dstv.at[j1]], ss1, add=True)
                return 0

            lax.fori_loop(1, nb // 2, rb, 0)
            pltpu.make_async_copy(rows0, acc.at[dstv.at[0]], ss0).wait()
            pltpu.make_async_copy(rows1, acc.at[dstv.at[0]], ss1).wait()
            plsc.subcore_barrier()

            m_cnt = _my_blocks(sid, nzt)

            def ob(j, _):
                blk = sid + j * NS
                for k in range(8):
                    start = (blk * 128 + k * 16) * CH + c
                    idxv[j, pl.ds(k * 16, 16)] = start + ramp8
                return 0

            lax.fori_loop(0, m_cnt, ob, 0)

            def co_issue(j, rows, sem):
                blk = sid + j * NS
                pltpu.sync_copy(acc.at[pl.ds(blk * 128, 128)], rows)
                pltpu.async_copy(rows, out.at[idxv.at[j]], sem)

            co_issue(0, rows0, ss0)
            co_issue(1, rows1, ss1)

            def cb(jj, _):
                pltpu.make_async_copy(rows0, out.at[idxv.at[0]], ss0).wait()
                co_issue(2 * jj, rows0, ss0)
                pltpu.make_async_copy(rows1, out.at[idxv.at[0]], ss1).wait()
                co_issue(2 * jj + 1, rows1, ss1)
                return 0

            lax.fori_loop(1, m_cnt // 2, cb, 0)
            pltpu.make_async_copy(rows0, out.at[idxv.at[0]], ss0).wait()
            pltpu.make_async_copy(rows1, out.at[idxv.at[0]], ss1).wait()

            @pl.when(m_cnt % 2 == 1)
            def _():
                co_issue(m_cnt - 1, rows0, ss0)
                pltpu.make_async_copy(rows0, out.at[idxv.at[0]], ss0).wait()
            return 0

        lax.fori_loop(0, CH // NC, chunk_body, 0)


def kernel(x_clause, x_literal, x_term, x_symbol, x_variable, params,
           ei_contains_literal, ei_has_atom, ei_has_arg, ei_symbol_of,
           ei_var_occurrence, ei_shared_variable):
    xs = {"clause": x_clause, "literal": x_literal, "term": x_term,
          "symbol": x_symbol, "variable": x_variable}
    eis = {"contains_literal": ei_contains_literal, "has_atom": ei_has_atom,
           "has_arg": ei_has_arg, "symbol_of": ei_symbol_of,
           "var_occurrence": ei_var_occurrence,
           "shared_variable": ei_shared_variable}

    x = {nt: _in_proj(xs[nt], *params["in_proj"][nt]) for nt in NN}
    sym_ids = jnp.clip(xs["symbol"][:, 0].astype(jnp.int32), 0, SYMV - 1)

    meta, ei_args, dst2_list = [], [], []
    for name, swap, snt, dnt, rel in ETYPES:
        ei = eis[name]
        src, dst = (ei[1], ei[0]) if swap else (ei[0], ei[1])
        e = src.shape[0]
        epad = _rup(e, 4096)
        nb = epad // (128 * NS)
        src2 = jnp.pad(src, (0, epad - e)).reshape(NS, nb, 128)
        dst2 = jnp.pad(dst, (0, epad - e),
                       constant_values=NN[dnt]).reshape(NS, nb, 128)
        meta.append({"nb": nb, "np_pad": _rup(NN[dnt] + 1, 128),
                     "src_i": NTS.index(snt), "dnt": dnt, "rel": rel})
        ei_args += [src2, dst2]
        dst2_list.append(dst2)
    np_max = max(m["np_pad"] for m in meta)
    nb_max = max(m["nb"] for m in meta)
    mesh = plsc.VectorSubcoreMesh(core_axis_name="c", subcore_axis_name="s",
                                  num_cores=NC, num_subcores=NS)
    sc_params = pltpu.CompilerParams(use_tc_tiling_on_sc=False)
    f32 = jnp.float32

    n_emb_rows = _rup(SYMV, 128 * NS) // 128
    sym_pad = jnp.pad(sym_ids, (0, n_emb_rows * 128 - SYMV)).reshape(
        NS, n_emb_rows // NS, 128)
    prepass = pl.kernel(
        functools.partial(_sc_prepass_body, meta, n_emb_rows),
        out_type=[jax.ShapeDtypeStruct((n_emb_rows * 128, SYMD), f32)]
        + [jax.ShapeDtypeStruct((m["np_pad"], CW), f32) for m in meta],
        mesh=mesh,
        compiler_params=sc_params,
        scratch_types=[
            pltpu.VMEM_SHARED((np_max, CW), f32),
            pltpu.VMEM((nb_max, 128), jnp.int32),
            pltpu.VMEM((128, CW), f32),
            pltpu.VMEM((128, CW), f32),
            pltpu.VMEM((128, SYMD), f32),
            pltpu.SemaphoreType.DMA,
            pltpu.SemaphoreType.DMA,
            pltpu.SemaphoreType.DMA,
        ],
    )
    emb_full, *cnts_full = prepass(params["sym_table"], sym_pad, *dst2_list)
    emb = emb_full[:SYMV]
    cnts = {m["rel"]: cnts_full[t][: NN[m["dnt"]], :1]
            for t, m in enumerate(meta)}

    x["symbol"] = _symcomb(x["symbol"], emb, *params["sym_comb"])

    layer_call = pl.kernel(
        functools.partial(_sc_layer_body, meta),
        out_type=[jax.ShapeDtypeStruct((m["np_pad"] * CH, CW), f32)
                  for m in meta],
        mesh=mesh,
        compiler_params=sc_params,
        scratch_types=[
            pltpu.VMEM_SHARED((np_max, CW), f32),
            pltpu.VMEM((nb_max, 128), jnp.int32),
            pltpu.VMEM((nb_max, 128), jnp.int32),
            pltpu.VMEM((nb_max, 128), jnp.int32),
            pltpu.VMEM((128, CW), f32),
            pltpu.VMEM((128, CW), f32),
            pltpu.VMEM((128, CW), f32),
            pltpu.SemaphoreType.DMA,
            pltpu.SemaphoreType.DMA,
            pltpu.SemaphoreType.DMA,
            pltpu.SemaphoreType.DMA,
        ],
    )

    for layer in params["layers"]:
        x8s = [x[nt].reshape(NN[nt] * CH, CW) for nt in NTS]
        s_all = layer_call(*ei_args, *x8s)
        s_by_dst = {nt: [] for nt in NN}
        for t, m in enumerate(meta):
            s = s_all[t].reshape(m["np_pad"], HID)
            s_by_dst[m["dnt"]].append((m["rel"], s))
        newx = {}
        for nt in NN:
            items = s_by_dst[nt]
            s_list = [s for _, s in items]
            cnt_list = [cnts[rel] for rel, _ in items]
            wl = jnp.stack([layer["convs"][rel][0] for rel, _ in items])
            bl = jnp.stack([layer["convs"][rel][1] for rel, _ in items])
            wr = jnp.stack([layer["convs"][rel][2] for rel, _ in items])
            g, b = layer["norms"][nt]
            newx[nt] = _update(x[nt], s_list, cnt_list, wl, wr, bl, g, b)
        x = newx

    return _head(x["clause"], *params["out1"], *params["out2"])


# speedup vs baseline: 1.0856x; 1.0856x over previous
import functools

import jax
import jax.numpy as jnp
from jax import lax
from jax.experimental import pallas as pl
from jax.experimental.pallas import tpu as pltpu
from jax.experimental.pallas import tpu_sc as plsc

NC, NS = 2, 16
CH, CW = 16, 16

HID = 256
EMBD = 512
SYMV = 10000
SYMD = 64
NFEAT = {"clause": 7, "literal": 3, "term": 8, "symbol": 6, "variable": 1}
NN = {"clause": 10000, "literal": 30000, "term": 60000, "symbol": 10000, "variable": 15000}
ETYPES = [
    ("contains_literal", False, "clause", "literal", "contains_literal"),
    ("has_atom", False, "literal", "term", "has_atom"),
    ("has_arg", False, "term", "term", "has_arg"),
    ("symbol_of", False, "term", "symbol", "symbol_of"),
    ("var_occurrence", False, "variable", "term", "var_occurrence"),
    ("shared_variable", False, "variable", "variable", "shared_variable"),
    ("contains_literal", True, "literal", "clause", "rev_contains_literal"),
    ("has_atom", True, "term", "literal", "rev_has_atom"),
    ("has_arg", True, "term", "term", "rev_has_arg"),
    ("symbol_of", True, "symbol", "term", "rev_symbol_of"),
    ("var_occurrence", True, "term", "variable", "rev_var_occurrence"),
]
RBLK = 1000


def _proj_body(x_ref, w_ref, b_ref, o_ref):
    o_ref[...] = jax.nn.relu(
        jnp.dot(x_ref[...], w_ref[...], preferred_element_type=jnp.float32)
        + b_ref[...]
    )


def _in_proj(x, w, b):
    n, f = x.shape
    fp = 8
    xp = jnp.pad(x, ((0, 0), (0, fp - f)))
    wp = jnp.pad(w, ((0, fp - f), (0, 0)))
    return pl.pallas_call(
        _proj_body,
        grid=(n // RBLK,),
        in_specs=[
            pl.BlockSpec((RBLK, fp), lambda i: (i, 0)),
            pl.BlockSpec((fp, HID), lambda i: (0, 0)),
            pl.BlockSpec((1, HID), lambda i: (0, 0)),
        ],
        out_specs=pl.BlockSpec((RBLK, HID), lambda i: (i, 0)),
        out_shape=jax.ShapeDtypeStruct((n, HID), jnp.float32),
    )(xp, wp, b.reshape(1, HID))


def _symcomb_body(x_ref, e_ref, w1_ref, w2_ref, b_ref, o_ref):
    o_ref[...] = jax.nn.relu(
        jnp.dot(x_ref[...], w1_ref[...], preferred_element_type=jnp.float32)
        + jnp.dot(e_ref[...], w2_ref[...], preferred_element_type=jnp.float32)
        + b_ref[...]
    )


def _symcomb(x, emb, wc, bc):
    n = x.shape[0]
    return pl.pallas_call(
        _symcomb_body,
        grid=(n // RBLK,),
        in_specs=[
            pl.BlockSpec((RBLK, HID), lambda i: (i, 0)),
            pl.BlockSpec((RBLK, SYMD), lambda i: (i, 0)),
            pl.BlockSpec((HID, HID), lambda i: (0, 0)),
            pl.BlockSpec((SYMD, HID), lambda i: (0, 0)),
            pl.BlockSpec((1, HID), lambda i: (0, 0)),
        ],
        out_specs=pl.BlockSpec((RBLK, HID), lambda i: (i, 0)),
        out_shape=jax.ShapeDtypeStruct((n, HID), jnp.float32),
    )(x, emb, wc[:HID], wc[HID:], bc.reshape(1, HID))


def _update_body(k, x_ref, *refs):
    s_refs = refs[:k]
    c_refs = refs[k : 2 * k]
    wl_ref, wr_ref, bl_ref, g_ref, b_ref, o_ref = refs[2 * k :]
    xb = x_ref[...]
    wr_sum = jnp.sum(wr_ref[...], axis=0)
    o = jnp.dot(xb, wr_sum, preferred_element_type=jnp.float32)
    o = o + jnp.sum(bl_ref[...], axis=0)[None, :]
    for i in range(k):
        ic = 1.0 / jnp.maximum(c_refs[i][...], 1.0)
        o = o + jnp.dot(
            s_refs[i][...] * ic, wl_ref[i], preferred_element_type=jnp.float32
        )
    h = o + xb
    m = jnp.mean(h, axis=-1, keepdims=True)
    v = jnp.mean((h - m) ** 2, axis=-1, keepdims=True)
    o_ref[...] = (h - m) * jax.lax.rsqrt(v + 1e-5) * g_ref[...] + b_ref[...]


def _update(x, s_list, cnt_list, wl, wr, bl, g, b):
    n = x.shape[0]
    k = len(s_list)
    in_specs = [pl.BlockSpec((RBLK, HID), lambda i: (i, 0))]
    in_specs += [pl.BlockSpec((RBLK, HID), lambda i: (i, 0))] * k
    in_specs += [pl.BlockSpec((RBLK, 1), lambda i: (i, 0))] * k
    in_specs += [
        pl.BlockSpec((k, HID, HID), lambda i: (0, 0, 0)),
        pl.BlockSpec((k, HID, HID), lambda i: (0, 0, 0)),
        pl.BlockSpec((k, HID), lambda i: (0, 0)),
        pl.BlockSpec((1, HID), lambda i: (0, 0)),
        pl.BlockSpec((1, HID), lambda i: (0, 0)),
    ]
    return pl.pallas_call(
        functools.partial(_update_body, k),
        grid=(n // RBLK,),
        in_specs=in_specs,
        out_specs=pl.BlockSpec((RBLK, HID), lambda i: (i, 0)),
        out_shape=jax.ShapeDtypeStruct((n, HID), jnp.float32),
    )(x, *s_list, *cnt_list, wl, wr, bl, g.reshape(1, HID), b.reshape(1, HID))


def _head_body(x_ref, w1_ref, b1_ref, w2_ref, b2_ref, o_ref):
    h = jax.nn.relu(
        jnp.dot(x_ref[...], w1_ref[...], preferred_element_type=jnp.float32)
        + b1_ref[...]
    )
    o_ref[...] = (
        jnp.dot(h, w2_ref[...], preferred_element_type=jnp.float32) + b2_ref[...]
    )


def _head(x, w1, b1, w2, b2):
    n = x.shape[0]
    return pl.pallas_call(
        _head_body,
        grid=(n // RBLK,),
        in_specs=[
            pl.BlockSpec((RBLK, HID), lambda i: (i, 0)),
            pl.BlockSpec((HID, HID), lambda i: (0, 0)),
            pl.BlockSpec((1, HID), lambda i: (0, 0)),
            pl.BlockSpec((HID, EMBD), lambda i: (0, 0)),
            pl.BlockSpec((1, EMBD), lambda i: (0, 0)),
        ],
        out_specs=pl.BlockSpec((RBLK, EMBD), lambda i: (i, 0)),
        out_shape=jax.ShapeDtypeStruct((n, EMBD), jnp.float32),
    )(x, w1, b1.reshape(1, HID), w2, b2.reshape(1, EMBD))


def _rup(x, m):
    return (x + m - 1) // m * m


NTS = list(NN)


def _zero_fill(buf, val):
    def zf(i, _):
        for k in range(CW // 16):
            buf[i, pl.ds(k * 16, 16)] = jnp.full((16,), val, jnp.float32)
        return 0

    lax.fori_loop(0, 128, zf, 0)


def _my_blocks(sid, nzt):
    return (nzt - sid + NS - 1) // NS


def _fill_rows(src_v, dst_ref, sid, nzt):
    def zb(j, _):
        blk = sid + j * NS
        pltpu.sync_copy(src_v, dst_ref.at[pl.ds(blk * 128, 128)])
        return 0

    lax.fori_loop(0, _my_blocks(sid, nzt), zb, 0)


def _sc_prepass_body(meta, n_emb_rows, *refs):
    num_et = len(meta)
    table, symidx = refs[0], refs[1]
    dst2 = refs[2 : 2 + num_et]
    emb_out = refs[2 + num_et]
    cnt_out = refs[3 + num_et : 3 + 2 * num_et]
    acc, dstv, zerosv, onesv, rows64, sg, ss0, ss1 = refs[3 + 2 * num_et :]
    cid = lax.axis_index("c")
    sid = lax.axis_index("s")
    _zero_fill(zerosv, 0.0)
    _zero_fill(onesv, 1.0)

    @pl.when(cid == 0)
    def _():
        rpt = n_emb_rows // NS
        pltpu.sync_copy(symidx.at[sid], dstv.at[pl.ds(0, rpt)])

        def eb(j, _):
            r = sid * rpt + j
            pltpu.async_copy(table.at[dstv.at[j]], rows64, sg).wait()
            pltpu.sync_copy(rows64, emb_out.at[pl.ds(r * 128, 128)])
            return 0

        lax.fori_loop(0, rpt, eb, 0)

    for t in range(num_et):
        m = meta[t]
        nb, npp = m["nb"], m["np_pad"]
        nzt = npp // 128

        @pl.when(cid == t % 2)
        def _(t=t, nb=nb, nzt=nzt):
            _fill_rows(zerosv, acc, sid, nzt)
            pltpu.sync_copy(dst2[t].at[sid], dstv.at[pl.ds(0, nb)])
            plsc.subcore_barrier()
            pltpu.async_copy(onesv, acc.at[dstv.at[0]], ss0, add=True)
            pltpu.async_copy(onesv, acc.at[dstv.at[1]], ss1, add=True)

            def rb(jj, _):
                pltpu.make_async_copy(onesv, acc.at[dstv.at[0]], ss0).wait()
                pltpu.async_copy(onesv, acc.at[dstv.at[2 * jj]], ss0, add=True)
                pltpu.make_async_copy(onesv, acc.at[dstv.at[0]], ss1).wait()
                pltpu.async_copy(
                    onesv, acc.at[dstv.at[2 * jj + 1]], ss1, add=True
                )
                return 0

            lax.fori_loop(1, nb // 2, rb, 0)
            pltpu.make_async_copy(onesv, acc.at[dstv.at[0]], ss0).wait()
            pltpu.make_async_copy(onesv, acc.at[dstv.at[0]], ss1).wait()
            plsc.subcore_barrier()

            def co(j, _):
                r0 = (sid + j * NS) * 128
                pltpu.sync_copy(
                    acc.at[pl.ds(r0, 128)], cnt_out[t].at[pl.ds(r0, 128)]
                )
                return 0

            lax.fori_loop(0, _my_blocks(sid, nzt), co, 0)


def _sc_layer_body(meta, *refs):
    num_et = len(meta)
    ei_refs = refs[: 2 * num_et]
    x8 = refs[2 * num_et : 2 * num_et + len(NTS)]
    outs = refs[2 * num_et + len(NTS) : 3 * num_et + len(NTS)]
    (acc, srcv, dstv, idxv, rows0, rows1, zerosv,
     sg0, sg1, ss0, ss1) = refs[3 * num_et + len(NTS) :]
    cid = lax.axis_index("c")
    sid = lax.axis_index("s")
    _zero_fill(zerosv, 0.0)

    ramp8 = lax.iota(jnp.int32, 16) * CH

    for t in range(num_et):
        m = meta[t]
        nb, npp = m["nb"], m["np_pad"]
        src2, dst2 = ei_refs[2 * t], ei_refs[2 * t + 1]
        xsrc = x8[m["src_i"]]
        out = outs[t]
        nzt = npp // 128
        pltpu.sync_copy(src2.at[sid], srcv.at[pl.ds(0, nb)])
        pltpu.sync_copy(dst2.at[sid], dstv.at[pl.ds(0, nb)])

        def chunk_body(cc, _, nb=nb, nzt=nzt, xsrc=xsrc, out=out):
            c = cid * (CH // NC) + cc
            _fill_rows(zerosv, acc, sid, nzt)

            def ib(j, _):
                for k in range(8):
                    v = srcv[j, pl.ds(k * 16, 16)]
                    idxv[j, pl.ds(k * 16, 16)] = v * CH + c
                return 0

            lax.fori_loop(0, nb, ib, 0)
            plsc.subcore_barrier()

            def g_issue(j, half, rows, sem):
                pltpu.async_copy(
                    xsrc.at[idxv.at[j]], rows.at[pl.ds(half * 128, 128)], sem
                )

            def s_issue(j, half, rows, sem):
                pltpu.async_copy(
                    rows.at[pl.ds(half * 128, 128)], acc.at[dstv.at[j]],
                    sem, add=True,
                )

            def g_wait(rows, sem):
                pltpu.make_async_copy(
                    xsrc.at[idxv.at[0]], rows.at[pl.ds(0, 128)], sem
                ).wait()

            def s_wait(rows, sem):
                pltpu.make_async_copy(
                    rows.at[pl.ds(0, 128)], acc.at[dstv.at[0]], sem
                ).wait()

            g_issue(0, 0, rows0, sg0)
            g_issue(1, 1, rows0, sg0)
            g_issue(2, 0, rows1, sg1)
            g_issue(3, 1, rows1, sg1)
            g_wait(rows0, sg0)
            g_wait(rows0, sg0)
            s_issue(0, 0, rows0, ss0)
            s_issue(1, 1, rows0, ss0)
            g_wait(rows1, sg1)
            g_wait(rows1, sg1)
            s_issue(2, 0, rows1, ss1)
            s_issue(3, 1, rows1, ss1)

            def rb(jj, _):
                j = 4 * jj
                s_wait(rows0, ss0)
                s_wait(rows0, ss0)
                g_issue(j, 0, rows0, sg0)
                g_issue(j + 1, 1, rows0, sg0)
                s_wait(rows1, ss1)
                s_wait(rows1, ss1)
                g_issue(j + 2, 0, rows1, sg1)
                g_issue(j + 3, 1, rows1, sg1)
                g_wait(rows0, sg0)
                g_wait(rows0, sg0)
                s_issue(j, 0, rows0, ss0)
                s_issue(j + 1, 1, rows0, ss0)
                g_wait(rows1, sg1)
                g_wait(rows1, sg1)
                s_issue(j + 2, 0, rows1, ss1)
                s_issue(j + 3, 1, rows1, ss1)
                return 0

            lax.fori_loop(1, nb // 4, rb, 0)
            s_wait(rows0, ss0)
            s_wait(rows0, ss0)
            s_wait(rows1, ss1)
            s_wait(rows1, ss1)
            if nb % 4:
                j = nb - 2
                g_issue(j, 0, rows0, sg0)
                g_issue(j + 1, 1, rows0, sg0)
                g_wait(rows0, sg0)
                g_wait(rows0, sg0)
                s_issue(j, 0, rows0, ss0)
                s_issue(j + 1, 1, rows0, ss0)
                s_wait(rows0, ss0)
                s_wait(rows0, ss0)
            plsc.subcore_barrier()

            m_cnt = _my_blocks(sid, nzt)

            def ob(j, _):
                blk = sid + j * NS
                for k in range(8):
                    start = (blk * 128 + k * 16) * CH + c
                    idxv[j, pl.ds(k * 16, 16)] = start + ramp8
                return 0

            lax.fori_loop(0, m_cnt, ob, 0)

            def co_issue(j, rows, sem):
                blk = sid + j * NS
                pltpu.sync_copy(
                    acc.at[pl.ds(blk * 128, 128)], rows.at[pl.ds(0, 128)]
                )
                pltpu.async_copy(rows.at[pl.ds(0, 128)], out.at[idxv.at[j]], sem)

            co_issue(0, rows0, ss0)
            co_issue(1, rows1, ss1)

            def cb(jj, _):
                pltpu.make_async_copy(rows0.at[pl.ds(0, 128)], out.at[idxv.at[0]], ss0).wait()
                co_issue(2 * jj, rows0, ss0)
                pltpu.make_async_copy(rows1.at[pl.ds(0, 128)], out.at[idxv.at[0]], ss1).wait()
                co_issue(2 * jj + 1, rows1, ss1)
                return 0

            lax.fori_loop(1, m_cnt // 2, cb, 0)
            pltpu.make_async_copy(rows0.at[pl.ds(0, 128)], out.at[idxv.at[0]], ss0).wait()
            pltpu.make_async_copy(rows1.at[pl.ds(0, 128)], out.at[idxv.at[0]], ss1).wait()

            @pl.when(m_cnt % 2 == 1)
            def _():
                co_issue(m_cnt - 1, rows0, ss0)
                pltpu.make_async_copy(rows0.at[pl.ds(0, 128)], out.at[idxv.at[0]], ss0).wait()
            return 0

        lax.fori_loop(0, CH // NC, chunk_body, 0)


def kernel(x_clause, x_literal, x_term, x_symbol, x_variable, params,
           ei_contains_literal, ei_has_atom, ei_has_arg, ei_symbol_of,
           ei_var_occurrence, ei_shared_variable):
    xs = {"clause": x_clause, "literal": x_literal, "term": x_term,
          "symbol": x_symbol, "variable": x_variable}
    eis = {"contains_literal": ei_contains_literal, "has_atom": ei_has_atom,
           "has_arg": ei_has_arg, "symbol_of": ei_symbol_of,
           "var_occurrence": ei_var_occurrence,
           "shared_variable": ei_shared_variable}

    x = {nt: _in_proj(xs[nt], *params["in_proj"][nt]) for nt in NN}
    sym_ids = jnp.clip(xs["symbol"][:, 0].astype(jnp.int32), 0, SYMV - 1)

    meta, ei_args, dst2_list = [], [], []
    for name, swap, snt, dnt, rel in ETYPES:
        ei = eis[name]
        src, dst = (ei[1], ei[0]) if swap else (ei[0], ei[1])
        e = src.shape[0]
        epad = _rup(e, 4096)
        nb = epad // (128 * NS)
        src2 = jnp.pad(src, (0, epad - e)).reshape(NS, nb, 128)
        dst2 = jnp.pad(dst, (0, epad - e),
                       constant_values=NN[dnt]).reshape(NS, nb, 128)
        meta.append({"nb": nb, "np_pad": _rup(NN[dnt] + 1, 128),
                     "src_i": NTS.index(snt), "dnt": dnt, "rel": rel})
        ei_args += [src2, dst2]
        dst2_list.append(dst2)
    np_max = max(m["np_pad"] for m in meta)
    nb_max = max(m["nb"] for m in meta)
    mesh = plsc.VectorSubcoreMesh(core_axis_name="c", subcore_axis_name="s",
                                  num_cores=NC, num_subcores=NS)
    sc_params = pltpu.CompilerParams(use_tc_tiling_on_sc=False)
    f32 = jnp.float32

    n_emb_rows = _rup(SYMV, 128 * NS) // 128
    sym_pad = jnp.pad(sym_ids, (0, n_emb_rows * 128 - SYMV)).reshape(
        NS, n_emb_rows // NS, 128)
    prepass = pl.kernel(
        functools.partial(_sc_prepass_body, meta, n_emb_rows),
        out_type=[jax.ShapeDtypeStruct((n_emb_rows * 128, SYMD), f32)]
        + [jax.ShapeDtypeStruct((m["np_pad"], CW), f32) for m in meta],
        mesh=mesh,
        compiler_params=sc_params,
        scratch_types=[
            pltpu.VMEM_SHARED((np_max, CW), f32),
            pltpu.VMEM((nb_max, 128), jnp.int32),
            pltpu.VMEM((128, CW), f32),
            pltpu.VMEM((128, CW), f32),
            pltpu.VMEM((128, SYMD), f32),
            pltpu.SemaphoreType.DMA,
            pltpu.SemaphoreType.DMA,
            pltpu.SemaphoreType.DMA,
        ],
    )
    emb_full, *cnts_full = prepass(params["sym_table"], sym_pad, *dst2_list)
    emb = emb_full[:SYMV]
    cnts = {m["rel"]: cnts_full[t][: NN[m["dnt"]], :1]
            for t, m in enumerate(meta)}

    x["symbol"] = _symcomb(x["symbol"], emb, *params["sym_comb"])

    layer_call = pl.kernel(
        functools.partial(_sc_layer_body, meta),
        out_type=[jax.ShapeDtypeStruct((m["np_pad"] * CH, CW), f32)
                  for m in meta],
        mesh=mesh,
        compiler_params=sc_params,
        scratch_types=[
            pltpu.VMEM_SHARED((np_max, CW), f32),
            pltpu.VMEM((nb_max, 128), jnp.int32),
            pltpu.VMEM((nb_max, 128), jnp.int32),
            pltpu.VMEM((nb_max, 128), jnp.int32),
            pltpu.VMEM((256, CW), f32),
            pltpu.VMEM((256, CW), f32),
            pltpu.VMEM((128, CW), f32),
            pltpu.SemaphoreType.DMA,
            pltpu.SemaphoreType.DMA,
            pltpu.SemaphoreType.DMA,
            pltpu.SemaphoreType.DMA,
        ],
    )

    for layer in params["layers"]:
        x8s = [x[nt].reshape(NN[nt] * CH, CW) for nt in NTS]
        s_all = layer_call(*ei_args, *x8s)
        s_by_dst = {nt: [] for nt in NN}
        for t, m in enumerate(meta):
            s = s_all[t].reshape(m["np_pad"], HID)
            s_by_dst[m["dnt"]].append((m["rel"], s))
        newx = {}
        for nt in NN:
            items = s_by_dst[nt]
            s_list = [s for _, s in items]
            cnt_list = [cnts[rel] for rel, _ in items]
            wl = jnp.stack([layer["convs"][rel][0] for rel, _ in items])
            bl = jnp.stack([layer["convs"][rel][1] for rel, _ in items])
            wr = jnp.stack([layer["convs"][rel][2] for rel, _ in items])
            g, b = layer["norms"][nt]
            newx[nt] = _update(x[nt], s_list, cnt_list, wl, wr, bl, g, b)
        x = newx

    return _head(x["clause"], *params["out1"], *params["out2"])


# speedup vs baseline: 1.1154x; 1.0275x over previous
import functools

import jax
import jax.numpy as jnp
from jax import lax
from jax.experimental import pallas as pl
from jax.experimental.pallas import tpu as pltpu
from jax.experimental.pallas import tpu_sc as plsc

NC, NS = 2, 16
CH, CW = 16, 16

HID = 256
EMBD = 512
SYMV = 10000
SYMD = 64
NFEAT = {"clause": 7, "literal": 3, "term": 8, "symbol": 6, "variable": 1}
NN = {"clause": 10000, "literal": 30000, "term": 60000, "symbol": 10000, "variable": 15000}
ETYPES = [
    ("contains_literal", False, "clause", "literal", "contains_literal"),
    ("has_atom", False, "literal", "term", "has_atom"),
    ("has_arg", False, "term", "term", "has_arg"),
    ("symbol_of", False, "term", "symbol", "symbol_of"),
    ("var_occurrence", False, "variable", "term", "var_occurrence"),
    ("shared_variable", False, "variable", "variable", "shared_variable"),
    ("contains_literal", True, "literal", "clause", "rev_contains_literal"),
    ("has_atom", True, "term", "literal", "rev_has_atom"),
    ("has_arg", True, "term", "term", "rev_has_arg"),
    ("symbol_of", True, "symbol", "term", "rev_symbol_of"),
    ("var_occurrence", True, "term", "variable", "rev_var_occurrence"),
]
RBLK = 1000


def _proj_body(x_ref, w_ref, b_ref, o_ref):
    o_ref[...] = jax.nn.relu(
        jnp.dot(x_ref[...], w_ref[...], preferred_element_type=jnp.float32)
        + b_ref[...]
    )


def _in_proj(x, w, b):
    n, f = x.shape
    fp = 8
    xp = jnp.pad(x, ((0, 0), (0, fp - f)))
    wp = jnp.pad(w, ((0, fp - f), (0, 0)))
    return pl.pallas_call(
        _proj_body,
        grid=(n // RBLK,),
        in_specs=[
            pl.BlockSpec((RBLK, fp), lambda i: (i, 0)),
            pl.BlockSpec((fp, HID), lambda i: (0, 0)),
            pl.BlockSpec((1, HID), lambda i: (0, 0)),
        ],
        out_specs=pl.BlockSpec((RBLK, HID), lambda i: (i, 0)),
        out_shape=jax.ShapeDtypeStruct((n, HID), jnp.float32),
    )(xp, wp, b.reshape(1, HID))


def _symcomb_body(x_ref, e_ref, w1_ref, w2_ref, b_ref, o_ref):
    o_ref[...] = jax.nn.relu(
        jnp.dot(x_ref[...], w1_ref[...], preferred_element_type=jnp.float32)
        + jnp.dot(e_ref[...], w2_ref[...], preferred_element_type=jnp.float32)
        + b_ref[...]
    )


def _symcomb(x, emb, wc, bc):
    n = x.shape[0]
    return pl.pallas_call(
        _symcomb_body,
        grid=(n // RBLK,),
        in_specs=[
            pl.BlockSpec((RBLK, HID), lambda i: (i, 0)),
            pl.BlockSpec((RBLK, SYMD), lambda i: (i, 0)),
            pl.BlockSpec((HID, HID), lambda i: (0, 0)),
            pl.BlockSpec((SYMD, HID), lambda i: (0, 0)),
            pl.BlockSpec((1, HID), lambda i: (0, 0)),
        ],
        out_specs=pl.BlockSpec((RBLK, HID), lambda i: (i, 0)),
        out_shape=jax.ShapeDtypeStruct((n, HID), jnp.float32),
    )(x, emb, wc[:HID], wc[HID:], bc.reshape(1, HID))


def _update_body(k, x_ref, *refs):
    s_refs = refs[:k]
    c_refs = refs[k : 2 * k]
    wl_ref, wr_ref, bl_ref, g_ref, b_ref, o_ref = refs[2 * k :]
    xb = x_ref[...]
    wr_sum = jnp.sum(wr_ref[...], axis=0)
    o = jnp.dot(xb, wr_sum, preferred_element_type=jnp.float32)
    o = o + jnp.sum(bl_ref[...], axis=0)[None, :]
    for i in range(k):
        ic = 1.0 / jnp.maximum(c_refs[i][...], 1.0)
        o = o + jnp.dot(
            s_refs[i][...] * ic, wl_ref[i], preferred_element_type=jnp.float32
        )
    h = o + xb
    m = jnp.mean(h, axis=-1, keepdims=True)
    v = jnp.mean((h - m) ** 2, axis=-1, keepdims=True)
    o_ref[...] = (h - m) * jax.lax.rsqrt(v + 1e-5) * g_ref[...] + b_ref[...]


def _update(x, s_list, cnt_list, wl, wr, bl, g, b):
    n = x.shape[0]
    k = len(s_list)
    in_specs = [pl.BlockSpec((RBLK, HID), lambda i: (i, 0))]
    in_specs += [pl.BlockSpec((RBLK, HID), lambda i: (i, 0))] * k
    in_specs += [pl.BlockSpec((RBLK, 1), lambda i: (i, 0))] * k
    in_specs += [
        pl.BlockSpec((k, HID, HID), lambda i: (0, 0, 0)),
        pl.BlockSpec((k, HID, HID), lambda i: (0, 0, 0)),
        pl.BlockSpec((k, HID), lambda i: (0, 0)),
        pl.BlockSpec((1, HID), lambda i: (0, 0)),
        pl.BlockSpec((1, HID), lambda i: (0, 0)),
    ]
    return pl.pallas_call(
        functools.partial(_update_body, k),
        grid=(n // RBLK,),
        in_specs=in_specs,
        out_specs=pl.BlockSpec((RBLK, HID), lambda i: (i, 0)),
        out_shape=jax.ShapeDtypeStruct((n, HID), jnp.float32),
    )(x, *s_list, *cnt_list, wl, wr, bl, g.reshape(1, HID), b.reshape(1, HID))


def _head_body(x_ref, w1_ref, b1_ref, w2_ref, b2_ref, o_ref):
    h = jax.nn.relu(
        jnp.dot(x_ref[...], w1_ref[...], preferred_element_type=jnp.float32)
        + b1_ref[...]
    )
    o_ref[...] = (
        jnp.dot(h, w2_ref[...], preferred_element_type=jnp.float32) + b2_ref[...]
    )


def _head(x, w1, b1, w2, b2):
    n = x.shape[0]
    return pl.pallas_call(
        _head_body,
        grid=(n // RBLK,),
        in_specs=[
            pl.BlockSpec((RBLK, HID), lambda i: (i, 0)),
            pl.BlockSpec((HID, HID), lambda i: (0, 0)),
            pl.BlockSpec((1, HID), lambda i: (0, 0)),
            pl.BlockSpec((HID, EMBD), lambda i: (0, 0)),
            pl.BlockSpec((1, EMBD), lambda i: (0, 0)),
        ],
        out_specs=pl.BlockSpec((RBLK, EMBD), lambda i: (i, 0)),
        out_shape=jax.ShapeDtypeStruct((n, EMBD), jnp.float32),
    )(x, w1, b1.reshape(1, HID), w2, b2.reshape(1, EMBD))


def _rup(x, m):
    return (x + m - 1) // m * m


NTS = list(NN)


def _zero_fill(buf, val):
    def zf(i, _):
        for k in range(CW // 16):
            buf[i, pl.ds(k * 16, 16)] = jnp.full((16,), val, jnp.float32)
        return 0

    lax.fori_loop(0, 128, zf, 0)


def _my_blocks(sid, nzt):
    return (nzt - sid + NS - 1) // NS


def _fill_rows(src_v, dst_ref, sid, nzt, sem):
    def zb(j, _):
        blk = sid + j * NS
        pltpu.async_copy(src_v, dst_ref.at[pl.ds(blk * 128, 128)], sem)
        return 0

    lax.fori_loop(0, _my_blocks(sid, nzt), zb, 0)

    def zw(j, _):
        pltpu.make_async_copy(
            src_v, dst_ref.at[pl.ds(sid * 128, 128)], sem
        ).wait()
        return 0

    lax.fori_loop(0, _my_blocks(sid, nzt), zw, 0)


def _sc_prepass_body(meta, n_emb_rows, *refs):
    num_et = len(meta)
    table, symidx = refs[0], refs[1]
    dst2 = refs[2 : 2 + num_et]
    emb_out = refs[2 + num_et]
    cnt_out = refs[3 + num_et : 3 + 2 * num_et]
    acc, dstv, zerosv, onesv, rows64, sg, ss0, ss1 = refs[3 + 2 * num_et :]
    cid = lax.axis_index("c")
    sid = lax.axis_index("s")
    _zero_fill(zerosv, 0.0)
    _zero_fill(onesv, 1.0)

    @pl.when(cid == 0)
    def _():
        rpt = n_emb_rows // NS
        pltpu.sync_copy(symidx.at[sid], dstv.at[pl.ds(0, rpt)])

        def eb(j, _):
            r = sid * rpt + j
            pltpu.async_copy(table.at[dstv.at[j]], rows64, sg).wait()
            pltpu.sync_copy(rows64, emb_out.at[pl.ds(r * 128, 128)])
            return 0

        lax.fori_loop(0, rpt, eb, 0)

    for t in range(num_et):
        m = meta[t]
        nb, npp = m["nb"], m["np_pad"]
        nzt = npp // 128

        @pl.when(cid == t % 2)
        def _(t=t, nb=nb, nzt=nzt):
            _fill_rows(zerosv, acc, sid, nzt, ss0)
            pltpu.sync_copy(dst2[t].at[sid], dstv.at[pl.ds(0, nb)])
            plsc.subcore_barrier()
            pltpu.async_copy(onesv, acc.at[dstv.at[0]], ss0, add=True)
            pltpu.async_copy(onesv, acc.at[dstv.at[1]], ss1, add=True)

            def rb(jj, _):
                pltpu.make_async_copy(onesv, acc.at[dstv.at[0]], ss0).wait()
                pltpu.async_copy(onesv, acc.at[dstv.at[2 * jj]], ss0, add=True)
                pltpu.make_async_copy(onesv, acc.at[dstv.at[0]], ss1).wait()
                pltpu.async_copy(
                    onesv, acc.at[dstv.at[2 * jj + 1]], ss1, add=True
                )
                return 0

            lax.fori_loop(1, nb // 2, rb, 0)
            pltpu.make_async_copy(onesv, acc.at[dstv.at[0]], ss0).wait()
            pltpu.make_async_copy(onesv, acc.at[dstv.at[0]], ss1).wait()
            plsc.subcore_barrier()

            def co(j, _):
                r0 = (sid + j * NS) * 128
                pltpu.sync_copy(
                    acc.at[pl.ds(r0, 128)], cnt_out[t].at[pl.ds(r0, 128)]
                )
                return 0

            lax.fori_loop(0, _my_blocks(sid, nzt), co, 0)


def _sc_layer_body(meta, *refs):
    num_et = len(meta)
    ei_refs = refs[: 2 * num_et]
    x8 = refs[2 * num_et : 2 * num_et + len(NTS)]
    outs = refs[2 * num_et + len(NTS) : 3 * num_et + len(NTS)]
    (acc, srcv, dstv, idxv, rows0, rows1, zerosv,
     sg0, sg1, ss0, ss1) = refs[3 * num_et + len(NTS) :]
    cid = lax.axis_index("c")
    sid = lax.axis_index("s")
    _zero_fill(zerosv, 0.0)

    ramp8 = lax.iota(jnp.int32, 16) * CH

    for t in range(num_et):
        m = meta[t]
        nb, npp = m["nb"], m["np_pad"]
        src2, dst2 = ei_refs[2 * t], ei_refs[2 * t + 1]
        xsrc = x8[m["src_i"]]
        out = outs[t]
        nzt = npp // 128
        pltpu.sync_copy(src2.at[sid], srcv.at[pl.ds(0, nb)])
        pltpu.sync_copy(dst2.at[sid], dstv.at[pl.ds(0, nb)])

        def chunk_body(cc, _, nb=nb, nzt=nzt, xsrc=xsrc, out=out):
            c = cid * (CH // NC) + cc
            _fill_rows(zerosv, acc, sid, nzt, ss0)

            def ib(j, _):
                for k in range(8):
                    v = srcv[j, pl.ds(k * 16, 16)]
                    idxv[j, pl.ds(k * 16, 16)] = v * CH + c
                return 0

            lax.fori_loop(0, nb, ib, 0)
            plsc.subcore_barrier()

            def g_issue(j, half, rows, sem):
                pltpu.async_copy(
                    xsrc.at[idxv.at[j]], rows.at[pl.ds(half * 128, 128)], sem
                )

            def s_issue(j, half, rows, sem):
                pltpu.async_copy(
                    rows.at[pl.ds(half * 128, 128)], acc.at[dstv.at[j]],
                    sem, add=True,
                )

            def g_wait(rows, sem):
                pltpu.make_async_copy(
                    xsrc.at[idxv.at[0]], rows.at[pl.ds(0, 128)], sem
                ).wait()

            def s_wait(rows, sem):
                pltpu.make_async_copy(
                    rows.at[pl.ds(0, 128)], acc.at[dstv.at[0]], sem
                ).wait()

            g_issue(0, 0, rows0, sg0)
            g_issue(1, 1, rows0, sg0)
            g_issue(2, 0, rows1, sg1)
            g_issue(3, 1, rows1, sg1)
            g_wait(rows0, sg0)
            g_wait(rows0, sg0)
            s_issue(0, 0, rows0, ss0)
            s_issue(1, 1, rows0, ss0)
            g_wait(rows1, sg1)
            g_wait(rows1, sg1)
            s_issue(2, 0, rows1, ss1)
            s_issue(3, 1, rows1, ss1)

            def rb(jj, _):
                j = 4 * jj
                s_wait(rows0, ss0)
                s_wait(rows0, ss0)
                g_issue(j, 0, rows0, sg0)
                g_issue(j + 1, 1, rows0, sg0)
                s_wait(rows1, ss1)
                s_wait(rows1, ss1)
                g_issue(j + 2, 0, rows1, sg1)
                g_issue(j + 3, 1, rows1, sg1)
                g_wait(rows0, sg0)
                g_wait(rows0, sg0)
                s_issue(j, 0, rows0, ss0)
                s_issue(j + 1, 1, rows0, ss0)
                g_wait(rows1, sg1)
                g_wait(rows1, sg1)
                s_issue(j + 2, 0, rows1, ss1)
                s_issue(j + 3, 1, rows1, ss1)
                return 0

            lax.fori_loop(1, nb // 4, rb, 0)
            s_wait(rows0, ss0)
            s_wait(rows0, ss0)
            s_wait(rows1, ss1)
            s_wait(rows1, ss1)
            if nb % 4:
                j = nb - 2
                g_issue(j, 0, rows0, sg0)
                g_issue(j + 1, 1, rows0, sg0)
                g_wait(rows0, sg0)
                g_wait(rows0, sg0)
                s_issue(j, 0, rows0, ss0)
                s_issue(j + 1, 1, rows0, ss0)
                s_wait(rows0, ss0)
                s_wait(rows0, ss0)
            plsc.subcore_barrier()

            m_cnt = _my_blocks(sid, nzt)

            def ob(j, _):
                blk = sid + j * NS
                for k in range(8):
                    start = (blk * 128 + k * 16) * CH + c
                    idxv[j, pl.ds(k * 16, 16)] = start + ramp8
                return 0

            lax.fori_loop(0, m_cnt, ob, 0)

            def co_issue(j, rows, sem):
                blk = sid + j * NS
                pltpu.sync_copy(
                    acc.at[pl.ds(blk * 128, 128)], rows.at[pl.ds(0, 128)]
                )
                pltpu.async_copy(rows.at[pl.ds(0, 128)], out.at[idxv.at[j]], sem)

            co_issue(0, rows0, ss0)
            co_issue(1, rows1, ss1)

            def cb(jj, _):
                pltpu.make_async_copy(rows0.at[pl.ds(0, 128)], out.at[idxv.at[0]], ss0).wait()
                co_issue(2 * jj, rows0, ss0)
                pltpu.make_async_copy(rows1.at[pl.ds(0, 128)], out.at[idxv.at[0]], ss1).wait()
                co_issue(2 * jj + 1, rows1, ss1)
                return 0

            lax.fori_loop(1, m_cnt // 2, cb, 0)
            pltpu.make_async_copy(rows0.at[pl.ds(0, 128)], out.at[idxv.at[0]], ss0).wait()
            pltpu.make_async_copy(rows1.at[pl.ds(0, 128)], out.at[idxv.at[0]], ss1).wait()

            @pl.when(m_cnt % 2 == 1)
            def _():
                co_issue(m_cnt - 1, rows0, ss0)
                pltpu.make_async_copy(rows0.at[pl.ds(0, 128)], out.at[idxv.at[0]], ss0).wait()
            return 0

        lax.fori_loop(0, CH // NC, chunk_body, 0)


def kernel(x_clause, x_literal, x_term, x_symbol, x_variable, params,
           ei_contains_literal, ei_has_atom, ei_has_arg, ei_symbol_of,
           ei_var_occurrence, ei_shared_variable):
    xs = {"clause": x_clause, "literal": x_literal, "term": x_term,
          "symbol": x_symbol, "variable": x_variable}
    eis = {"contains_literal": ei_contains_literal, "has_atom": ei_has_atom,
           "has_arg": ei_has_arg, "symbol_of": ei_symbol_of,
           "var_occurrence": ei_var_occurrence,
           "shared_variable": ei_shared_variable}

    x = {nt: _in_proj(xs[nt], *params["in_proj"][nt]) for nt in NN}
    sym_ids = jnp.clip(xs["symbol"][:, 0].astype(jnp.int32), 0, SYMV - 1)

    meta, ei_args, dst2_list = [], [], []
    for name, swap, snt, dnt, rel in ETYPES:
        ei = eis[name]
        src, dst = (ei[1], ei[0]) if swap else (ei[0], ei[1])
        e = src.shape[0]
        epad = _rup(e, 4096)
        nb = epad // (128 * NS)
        src2 = jnp.pad(src, (0, epad - e)).reshape(NS, nb, 128)
        dst2 = jnp.pad(dst, (0, epad - e),
                       constant_values=NN[dnt]).reshape(NS, nb, 128)
        meta.append({"nb": nb, "np_pad": _rup(NN[dnt] + 1, 128),
                     "src_i": NTS.index(snt), "dnt": dnt, "rel": rel})
        ei_args += [src2, dst2]
        dst2_list.append(dst2)
    np_max = max(m["np_pad"] for m in meta)
    nb_max = max(m["nb"] for m in meta)
    mesh = plsc.VectorSubcoreMesh(core_axis_name="c", subcore_axis_name="s",
                                  num_cores=NC, num_subcores=NS)
    sc_params = pltpu.CompilerParams(use_tc_tiling_on_sc=False)
    f32 = jnp.float32

    n_emb_rows = _rup(SYMV, 128 * NS) // 128
    sym_pad = jnp.pad(sym_ids, (0, n_emb_rows * 128 - SYMV)).reshape(
        NS, n_emb_rows // NS, 128)
    prepass = pl.kernel(
        functools.partial(_sc_prepass_body, meta, n_emb_rows),
        out_type=[jax.ShapeDtypeStruct((n_emb_rows * 128, SYMD), f32)]
        + [jax.ShapeDtypeStruct((m["np_pad"], CW), f32) for m in meta],
        mesh=mesh,
        compiler_params=sc_params,
        scratch_types=[
            pltpu.VMEM_SHARED((np_max, CW), f32),
            pltpu.VMEM((nb_max, 128), jnp.int32),
            pltpu.VMEM((128, CW), f32),
            pltpu.VMEM((128, CW), f32),
            pltpu.VMEM((128, SYMD), f32),
            pltpu.SemaphoreType.DMA,
            pltpu.SemaphoreType.DMA,
            pltpu.SemaphoreType.DMA,
        ],
    )
    emb_full, *cnts_full = prepass(params["sym_table"], sym_pad, *dst2_list)
    emb = emb_full[:SYMV]
    cnts = {m["rel"]: cnts_full[t][: NN[m["dnt"]], :1]
            for t, m in enumerate(meta)}

    x["symbol"] = _symcomb(x["symbol"], emb, *params["sym_comb"])

    layer_call = pl.kernel(
        functools.partial(_sc_layer_body, meta),
        out_type=[jax.ShapeDtypeStruct((m["np_pad"] * CH, CW), f32)
                  for m in meta],
        mesh=mesh,
        compiler_params=sc_params,
        scratch_types=[
            pltpu.VMEM_SHARED((np_max, CW), f32),
            pltpu.VMEM((nb_max, 128), jnp.int32),
            pltpu.VMEM((nb_max, 128), jnp.int32),
            pltpu.VMEM((nb_max, 128), jnp.int32),
            pltpu.VMEM((256, CW), f32),
            pltpu.VMEM((256, CW), f32),
            pltpu.VMEM((128, CW), f32),
            pltpu.SemaphoreType.DMA,
            pltpu.SemaphoreType.DMA,
            pltpu.SemaphoreType.DMA,
            pltpu.SemaphoreType.DMA,
        ],
    )

    for layer in params["layers"]:
        x8s = [x[nt].reshape(NN[nt] * CH, CW) for nt in NTS]
        s_all = layer_call(*ei_args, *x8s)
        s_by_dst = {nt: [] for nt in NN}
        for t, m in enumerate(meta):
            s = s_all[t].reshape(m["np_pad"], HID)
            s_by_dst[m["dnt"]].append((m["rel"], s))
        newx = {}
        for nt in NN:
            items = s_by_dst[nt]
            s_list = [s for _, s in items]
            cnt_list = [cnts[rel] for rel, _ in items]
            wl = jnp.stack([layer["convs"][rel][0] for rel, _ in items])
            bl = jnp.stack([layer["convs"][rel][1] for rel, _ in items])
            wr = jnp.stack([layer["convs"][rel][2] for rel, _ in items])
            g, b = layer["norms"][nt]
            newx[nt] = _update(x[nt], s_list, cnt_list, wl, wr, bl, g, b)
        x = newx

    return _head(x["clause"], *params["out1"], *params["out2"])
